# baseline (speedup 1.0000x reference)
import functools

import jax
import jax.numpy as jnp
from jax import lax
from jax.experimental import pallas as pl
from jax.experimental.pallas import tpu as pltpu
from jax.experimental.pallas import tpu_sc as plsc

_LANES = 16


def _sc_partial_sums(hidden_states, num_seqs):
    tokens, hidden = hidden_states.shape
    seg = tokens // num_seqs
    half = seg // 2
    chunk = 32
    nbuf = 2
    nchunks = half // chunk
    groups = hidden // _LANES
    ru = 16

    mesh = plsc.VectorSubcoreMesh(core_axis_name="c", subcore_axis_name="s")

    @functools.partial(
        pl.kernel,
        mesh=mesh,
        out_type=jax.ShapeDtypeStruct((2 * num_seqs, hidden), jnp.float32),
        scratch_types=(
            [pltpu.VMEM((chunk, hidden), jnp.float32)] * nbuf
            + [pltpu.VMEM((hidden,), jnp.float32)]
            + [pltpu.SemaphoreType.DMA] * nbuf
        ),
    )
    def sc_sums(hs_hbm, out_hbm, *refs):
        bufs = refs[:nbuf]
        acc = refs[nbuf]
        sems = refs[nbuf + 1:2 * nbuf + 1]
        c = lax.axis_index("c")
        s = lax.axis_index("s")
        row0 = s * seg + c * half
        out_row = c * num_seqs + s

        for j in range(groups):
            acc[pl.ds(j * _LANES, _LANES)] = jnp.zeros((_LANES,), jnp.float32)

        for b in range(nbuf - 1):
            pltpu.async_copy(
                hs_hbm.at[pl.ds(row0 + b * chunk, chunk), :], bufs[b], sems[b])

        def accumulate(buf):
            def rowstep(i, _):
                base = i * ru
                for j in range(groups):
                    sl = pl.ds(j * _LANES, _LANES)
                    vs = [buf[base + k, sl] for k in range(ru)]
                    while len(vs) > 1:
                        vs = [vs[k] + vs[k + 1] for k in range(0, len(vs), 2)]
                    acc[sl] = acc[sl] + vs[0]
                return 0

            lax.fori_loop(0, chunk // ru, rowstep, 0)

        def outer(i, _):
            for b in range(nbuf):
                ch = i * nbuf + b
                cur, sem = bufs[b], sems[b]
                ahead = (b + nbuf - 1) % nbuf
                nxt, nsem = bufs[ahead], sems[ahead]

                @pl.when(ch + nbuf - 1 < nchunks)
                def _start_next():
                    pltpu.async_copy(
                        hs_hbm.at[pl.ds(row0 + (ch + nbuf - 1) * chunk, chunk), :],
                        nxt, nsem)

                pltpu.make_async_copy(
                    hs_hbm.at[pl.ds(row0 + ch * chunk, chunk), :], cur, sem
                ).wait()
                accumulate(cur)
            return 0

        lax.fori_loop(0, nchunks // nbuf, outer, 0)
        pltpu.sync_copy(acc, out_hbm.at[out_row])

    return sc_sums(hidden_states)


def _finalize_body(p_ref, lens_ref, o_ref):
    n = o_ref.shape[0]
    sums = p_ref[0:n, :] + p_ref[n:2 * n, :]
    pooled = sums / lens_ref[...]
    nrm = jnp.sqrt(jnp.sum(pooled * pooled, axis=1, keepdims=True))
    o_ref[...] = pooled / jnp.maximum(nrm, 1e-12)


def kernel(hidden_states, extend_seq_lens):
    n = extend_seq_lens.shape[0]
    hidden = hidden_states.shape[1]
    partials = _sc_partial_sums(hidden_states, n)
    lens2d = extend_seq_lens.astype(jnp.float32).reshape(n, 1)
    return pl.pallas_call(
        _finalize_body,
        out_shape=jax.ShapeDtypeStruct((n, hidden), jnp.float32),
    )(partials, lens2d)


# speedup vs baseline: 1.0392x; 1.0392x over previous
import functools

import jax
import jax.numpy as jnp
from jax import lax
from jax.experimental import pallas as pl
from jax.experimental.pallas import tpu as pltpu
from jax.experimental.pallas import tpu_sc as plsc

_LANES = 16


def _sc_partial_sums(hidden_states, num_seqs):
    tokens, hidden = hidden_states.shape
    seg = tokens // num_seqs
    half = seg // 2
    chunk = 16
    nbuf = 2
    nchunks = half // chunk
    groups = hidden // _LANES
    ru = 8

    mesh = plsc.VectorSubcoreMesh(core_axis_name="c", subcore_axis_name="s")

    @functools.partial(
        pl.kernel,
        mesh=mesh,
        out_type=jax.ShapeDtypeStruct((2 * num_seqs, hidden), jnp.float32),
        scratch_types=(
            [pltpu.VMEM((chunk, hidden), jnp.float32)] * nbuf
            + [pltpu.VMEM((hidden,), jnp.float32)]
            + [pltpu.SemaphoreType.DMA] * nbuf
        ),
    )
    def sc_sums(hs_hbm, out_hbm, *refs):
        bufs = refs[:nbuf]
        acc = refs[nbuf]
        sems = refs[nbuf + 1:2 * nbuf + 1]
        c = lax.axis_index("c")
        s = lax.axis_index("s")
        row0 = s * seg + c * half
        out_row = c * num_seqs + s

        for j in range(groups):
            acc[pl.ds(j * _LANES, _LANES)] = jnp.zeros((_LANES,), jnp.float32)

        for b in range(nbuf - 1):
            pltpu.async_copy(
                hs_hbm.at[pl.ds(row0 + b * chunk, chunk), :], bufs[b], sems[b])

        def accumulate(buf):
            for i in range(chunk // ru):
                base = i * ru
                for j in range(groups):
                    sl = pl.ds(j * _LANES, _LANES)
                    vs = [buf[base + k, sl] for k in range(ru)]
                    while len(vs) > 1:
                        vs = [vs[k] + vs[k + 1] for k in range(0, len(vs), 2)]
                    acc[sl] = acc[sl] + vs[0]

        def outer(i, _):
            for b in range(nbuf):
                ch = i * nbuf + b
                cur, sem = bufs[b], sems[b]
                ahead = (b + nbuf - 1) % nbuf
                nxt, nsem = bufs[ahead], sems[ahead]

                @pl.when(ch + nbuf - 1 < nchunks)
                def _start_next():
                    pltpu.async_copy(
                        hs_hbm.at[pl.ds(row0 + (ch + nbuf - 1) * chunk, chunk), :],
                        nxt, nsem)

                pltpu.make_async_copy(
                    hs_hbm.at[pl.ds(row0 + ch * chunk, chunk), :], cur, sem
                ).wait()
                accumulate(cur)
            return 0

        lax.fori_loop(0, nchunks // nbuf, outer, 0)
        pltpu.sync_copy(acc, out_hbm.at[out_row])

    return sc_sums(hidden_states)


def _finalize_body(p_ref, lens_ref, o_ref):
    n = o_ref.shape[0]
    sums = p_ref[0:n, :] + p_ref[n:2 * n, :]
    pooled = sums / lens_ref[...]
    nrm = jnp.sqrt(jnp.sum(pooled * pooled, axis=1, keepdims=True))
    o_ref[...] = pooled / jnp.maximum(nrm, 1e-12)


def kernel(hidden_states, extend_seq_lens):
    n = extend_seq_lens.shape[0]
    hidden = hidden_states.shape[1]
    partials = _sc_partial_sums(hidden_states, n)
    lens2d = extend_seq_lens.astype(jnp.float32).reshape(n, 1)
    return pl.pallas_call(
        _finalize_body,
        out_shape=jax.ShapeDtypeStruct((n, hidden), jnp.float32),
    )(partials, lens2d)


# speedup vs baseline: 1.6600x; 1.5973x over previous
import functools

import jax
import jax.numpy as jnp
from jax import lax
from jax.experimental import pallas as pl
from jax.experimental.pallas import tpu as pltpu
from jax.experimental.pallas import tpu_sc as plsc

_LANES = 16


def _sc_partial_sums(hidden_states, num_seqs, sc_rows):
    tokens, hidden = hidden_states.shape
    seg = tokens // num_seqs
    half = sc_rows // 2
    chunk = 32
    nbuf = 2
    nchunks = half // chunk
    groups = hidden // _LANES
    ru = 8
    gbatch = 8

    mesh = plsc.VectorSubcoreMesh(core_axis_name="c", subcore_axis_name="s")

    @functools.partial(
        pl.kernel,
        mesh=mesh,
        out_type=jax.ShapeDtypeStruct((2 * num_seqs, hidden), jnp.float32),
        scratch_types=(
            [pltpu.VMEM((chunk, hidden), jnp.float32)] * nbuf
            + [pltpu.VMEM((hidden,), jnp.float32)]
            + [pltpu.SemaphoreType.DMA] * nbuf
        ),
    )
    def sc_sums(hs_hbm, out_hbm, *refs):
        bufs = refs[:nbuf]
        acc = refs[nbuf]
        sems = refs[nbuf + 1:2 * nbuf + 1]
        c = lax.axis_index("c")
        s = lax.axis_index("s")
        row0 = s * seg + c * half
        out_row = c * num_seqs + s

        for j in range(groups):
            acc[pl.ds(j * _LANES, _LANES)] = jnp.zeros((_LANES,), jnp.float32)

        for b in range(nbuf - 1):
            pltpu.async_copy(
                hs_hbm.at[pl.ds(row0 + b * chunk, chunk), :], bufs[b], sems[b])

        def accumulate(buf):
            def rowstep(i, _):
                base = i * ru
                for j0 in range(0, groups, gbatch):
                    temps = []
                    for j in range(j0, j0 + gbatch):
                        sl = pl.ds(j * _LANES, _LANES)
                        vs = [buf[base + k, sl] for k in range(ru)]
                        while len(vs) > 1:
                            vs = [vs[k] + vs[k + 1]
                                  for k in range(0, len(vs), 2)]
                        temps.append(vs[0])
                    for j, v in zip(range(j0, j0 + gbatch), temps):
                        plsc.addupdate(acc.at[pl.ds(j * _LANES, _LANES)], v)
                return 0

            lax.fori_loop(0, chunk // ru, rowstep, 0)

        def outer(i, _):
            for b in range(nbuf):
                ch = i * nbuf + b
                cur, sem = bufs[b], sems[b]
                ahead = (b + nbuf - 1) % nbuf
                nxt, nsem = bufs[ahead], sems[ahead]

                @pl.when(ch + nbuf - 1 < nchunks)
                def _start_next():
                    pltpu.async_copy(
                        hs_hbm.at[pl.ds(row0 + (ch + nbuf - 1) * chunk, chunk), :],
                        nxt, nsem)

                pltpu.make_async_copy(
                    hs_hbm.at[pl.ds(row0 + ch * chunk, chunk), :], cur, sem
                ).wait()
                accumulate(cur)
            return 0

        lax.fori_loop(0, nchunks // nbuf, outer, 0)
        pltpu.sync_copy(acc, out_hbm.at[out_row])

    return sc_sums(hidden_states)


def _tc_slab_body(x_ref, o_ref):
    i = pl.program_id(0)
    j = pl.program_id(1)
    blk_sum = jnp.sum(x_ref[...], axis=0, keepdims=True)

    @pl.when(j == 0)
    def _init():
        o_ref[pl.ds(i, 1), :] = blk_sum

    @pl.when(j > 0)
    def _acc():
        o_ref[pl.ds(i, 1), :] += blk_sum


def _tc_slab_sums(hidden_states, num_seqs, sc_rows, blk):
    tokens, hidden = hidden_states.shape
    seg = tokens // num_seqs
    m = (seg - sc_rows) // blk
    return pl.pallas_call(
        _tc_slab_body,
        grid=(num_seqs, m),
        in_specs=[pl.BlockSpec(
            (blk, hidden),
            lambda i, j: ((i * seg + sc_rows) // blk + j, 0))],
        out_specs=pl.BlockSpec((num_seqs, hidden), lambda i, j: (0, 0)),
        out_shape=jax.ShapeDtypeStruct((num_seqs, hidden), jnp.float32),
    )(hidden_states)


def _finalize_body(psc_ref, ptc_ref, lens_ref, o_ref):
    n = o_ref.shape[0]
    sums = psc_ref[0:n, :] + psc_ref[n:2 * n, :] + ptc_ref[...]
    pooled = sums / lens_ref[...]
    nrm = jnp.sqrt(jnp.sum(pooled * pooled, axis=1, keepdims=True))
    o_ref[...] = pooled / jnp.maximum(nrm, 1e-12)


def kernel(hidden_states, extend_seq_lens):
    n = extend_seq_lens.shape[0]
    tokens, hidden = hidden_states.shape
    seg = tokens // n
    blk = 128
    sc_rows = max(128, (seg * 3 // 16) // 128 * 128)
    partials_sc = _sc_partial_sums(hidden_states, n, sc_rows)
    partials_tc = _tc_slab_sums(hidden_states, n, sc_rows, blk)
    lens2d = extend_seq_lens.astype(jnp.float32).reshape(n, 1)
    return pl.pallas_call(
        _finalize_body,
        out_shape=jax.ShapeDtypeStruct((n, hidden), jnp.float32),
    )(partials_sc, partials_tc, lens2d)


# speedup vs baseline: 3.4293x; 2.0659x over previous
import functools

import jax
import jax.numpy as jnp
from jax import lax
from jax.experimental import pallas as pl
from jax.experimental.pallas import tpu as pltpu
from jax.experimental.pallas import tpu_sc as plsc

_LANES = 16


def _sc_partial_sums(hidden_states, num_seqs, sc_rows):
    tokens, hidden = hidden_states.shape
    seg = tokens // num_seqs
    half = sc_rows // 2
    chunk = 32
    nbuf = 2
    nchunks = half // chunk
    groups = hidden // _LANES
    ru = 8
    gbatch = 8

    mesh = plsc.VectorSubcoreMesh(core_axis_name="c", subcore_axis_name="s")

    @functools.partial(
        pl.kernel,
        mesh=mesh,
        out_type=jax.ShapeDtypeStruct((2 * num_seqs, hidden), jnp.float32),
        scratch_types=(
            [pltpu.VMEM((chunk, hidden), jnp.float32)] * nbuf
            + [pltpu.VMEM((hidden,), jnp.float32)]
            + [pltpu.SemaphoreType.DMA] * nbuf
        ),
    )
    def sc_sums(hs_hbm, out_hbm, *refs):
        bufs = refs[:nbuf]
        acc = refs[nbuf]
        sems = refs[nbuf + 1:2 * nbuf + 1]
        c = lax.axis_index("c")
        s = lax.axis_index("s")
        row0 = s * seg + c * half
        out_row = c * num_seqs + s

        for j in range(groups):
            acc[pl.ds(j * _LANES, _LANES)] = jnp.zeros((_LANES,), jnp.float32)

        for b in range(nbuf - 1):
            pltpu.async_copy(
                hs_hbm.at[pl.ds(row0 + b * chunk, chunk), :], bufs[b], sems[b])

        def accumulate(buf):
            def rowstep(i, _):
                base = i * ru
                for j0 in range(0, groups, gbatch):
                    temps = []
                    for j in range(j0, j0 + gbatch):
                        sl = pl.ds(j * _LANES, _LANES)
                        vs = [buf[base + k, sl] for k in range(ru)]
                        while len(vs) > 1:
                            vs = [vs[k] + vs[k + 1]
                                  for k in range(0, len(vs), 2)]
                        temps.append(vs[0])
                    for j, v in zip(range(j0, j0 + gbatch), temps):
                        plsc.addupdate(acc.at[pl.ds(j * _LANES, _LANES)], v)
                return 0

            lax.fori_loop(0, chunk // ru, rowstep, 0)

        def outer(i, _):
            for b in range(nbuf):
                ch = i * nbuf + b
                cur, sem = bufs[b], sems[b]
                ahead = (b + nbuf - 1) % nbuf
                nxt, nsem = bufs[ahead], sems[ahead]

                @pl.when(ch + nbuf - 1 < nchunks)
                def _start_next():
                    pltpu.async_copy(
                        hs_hbm.at[pl.ds(row0 + (ch + nbuf - 1) * chunk, chunk), :],
                        nxt, nsem)

                pltpu.make_async_copy(
                    hs_hbm.at[pl.ds(row0 + ch * chunk, chunk), :], cur, sem
                ).wait()
                accumulate(cur)
            return 0

        lax.fori_loop(0, nchunks // nbuf, outer, 0)
        pltpu.sync_copy(acc, out_hbm.at[out_row])

    return sc_sums(hidden_states)


def _tc_slab_body(x_ref, o_ref):
    i = pl.program_id(0)
    j = pl.program_id(1)
    blk_sum = jnp.sum(x_ref[...], axis=0, keepdims=True)

    @pl.when(j == 0)
    def _init():
        o_ref[pl.ds(i, 1), :] = blk_sum

    @pl.when(j > 0)
    def _acc():
        o_ref[pl.ds(i, 1), :] += blk_sum


def _tc_slab_sums(hidden_states, num_seqs, sc_rows, blk):
    tokens, hidden = hidden_states.shape
    seg = tokens // num_seqs
    m = (seg - sc_rows) // blk
    return pl.pallas_call(
        _tc_slab_body,
        grid=(num_seqs, m),
        in_specs=[pl.BlockSpec(
            (blk, hidden),
            lambda i, j: ((i * seg + sc_rows) // blk + j, 0))],
        out_specs=pl.BlockSpec((num_seqs, hidden), lambda i, j: (0, 0)),
        out_shape=jax.ShapeDtypeStruct((num_seqs, hidden), jnp.float32),
    )(hidden_states)


def _finalize_body(psc_ref, ptc_ref, lens_ref, o_ref):
    n = o_ref.shape[0]
    sums = psc_ref[0:n, :] + psc_ref[n:2 * n, :] + ptc_ref[...]
    pooled = sums / lens_ref[...]
    nrm = jnp.sqrt(jnp.sum(pooled * pooled, axis=1, keepdims=True))
    o_ref[...] = pooled / jnp.maximum(nrm, 1e-12)


def kernel(hidden_states, extend_seq_lens):
    n = extend_seq_lens.shape[0]
    tokens, hidden = hidden_states.shape
    seg = tokens // n
    blk = 512
    sc_rows = max(64, seg // 4 // 64 * 64)
    partials_sc = _sc_partial_sums(hidden_states, n, sc_rows)
    partials_tc = _tc_slab_sums(hidden_states, n, sc_rows, blk)
    lens2d = extend_seq_lens.astype(jnp.float32).reshape(n, 1)
    return pl.pallas_call(
        _finalize_body,
        out_shape=jax.ShapeDtypeStruct((n, hidden), jnp.float32),
    )(partials_sc, partials_tc, lens2d)


# speedup vs baseline: 3.8148x; 1.1124x over previous
import functools

import jax
import jax.numpy as jnp
from jax import lax
from jax.experimental import pallas as pl
from jax.experimental.pallas import tpu as pltpu
from jax.experimental.pallas import tpu_sc as plsc

_LANES = 16
_SUBCORES = 32


def _sc_partial_sums(hidden_states, seg, sc_segs):
    hidden = hidden_states.shape[1]
    per_seg = _SUBCORES // sc_segs
    half = seg // per_seg
    chunk = 32
    nbuf = 2
    nchunks = half // chunk
    groups = hidden // _LANES
    ru = 8
    gbatch = 8

    mesh = plsc.VectorSubcoreMesh(core_axis_name="c", subcore_axis_name="s")

    @functools.partial(
        pl.kernel,
        mesh=mesh,
        out_type=jax.ShapeDtypeStruct((_SUBCORES, hidden), jnp.float32),
        scratch_types=(
            [pltpu.VMEM((chunk, hidden), jnp.float32)] * nbuf
            + [pltpu.VMEM((hidden,), jnp.float32)]
            + [pltpu.SemaphoreType.DMA] * nbuf
        ),
    )
    def sc_sums(hs_hbm, out_hbm, *refs):
        bufs = refs[:nbuf]
        acc = refs[nbuf]
        sems = refs[nbuf + 1:2 * nbuf + 1]
        c = lax.axis_index("c")
        s = lax.axis_index("s")
        g = c * 16 + s
        row0 = (g // per_seg) * seg + (g % per_seg) * half

        for j in range(groups):
            acc[pl.ds(j * _LANES, _LANES)] = jnp.zeros((_LANES,), jnp.float32)

        for b in range(nbuf - 1):
            pltpu.async_copy(
                hs_hbm.at[pl.ds(row0 + b * chunk, chunk), :], bufs[b], sems[b])

        def accumulate(buf):
            def rowstep(i, _):
                base = i * ru
                for j0 in range(0, groups, gbatch):
                    temps = []
                    for j in range(j0, j0 + gbatch):
                        sl = pl.ds(j * _LANES, _LANES)
                        vs = [buf[base + k, sl] for k in range(ru)]
                        while len(vs) > 1:
                            vs = [vs[k] + vs[k + 1]
                                  for k in range(0, len(vs), 2)]
                        temps.append(vs[0])
                    for j, v in zip(range(j0, j0 + gbatch), temps):
                        plsc.addupdate(acc.at[pl.ds(j * _LANES, _LANES)], v)
                return 0

            lax.fori_loop(0, chunk // ru, rowstep, 0)

        def outer(i, _):
            for b in range(nbuf):
                ch = i * nbuf + b
                cur, sem = bufs[b], sems[b]
                ahead = (b + nbuf - 1) % nbuf
                nxt, nsem = bufs[ahead], sems[ahead]

                @pl.when(ch + nbuf - 1 < nchunks)
                def _start_next():
                    pltpu.async_copy(
                        hs_hbm.at[pl.ds(row0 + (ch + nbuf - 1) * chunk, chunk), :],
                        nxt, nsem)

                pltpu.make_async_copy(
                    hs_hbm.at[pl.ds(row0 + ch * chunk, chunk), :], cur, sem
                ).wait()
                accumulate(cur)
            return 0

        lax.fori_loop(0, nchunks // nbuf, outer, 0)
        pltpu.sync_copy(acc, out_hbm.at[g])

    return sc_sums(hidden_states)


def _tc_pool_body(lens_ref, x_ref, o_ref, *, sc_segs):
    i = pl.program_id(0)
    s = jnp.sum(x_ref[...], axis=0, keepdims=True)
    pooled = s / lens_ref[i + sc_segs].astype(jnp.float32)
    nrm = jnp.sqrt(jnp.sum(pooled * pooled))
    o_ref[pl.ds(i, 1), :] = pooled / jnp.maximum(nrm, 1e-12)


def _tc_pool(hidden_states, lens, seg, sc_segs):
    n = lens.shape[0]
    hidden = hidden_states.shape[1]
    m = n - sc_segs
    grid_spec = pltpu.PrefetchScalarGridSpec(
        num_scalar_prefetch=1,
        grid=(m,),
        in_specs=[pl.BlockSpec((seg, hidden), lambda i, lens: (i + sc_segs, 0))],
        out_specs=pl.BlockSpec((m, hidden), lambda i, lens: (0, 0)),
    )
    return pl.pallas_call(
        functools.partial(_tc_pool_body, sc_segs=sc_segs),
        grid_spec=grid_spec,
        out_shape=jax.ShapeDtypeStruct((m, hidden), jnp.float32),
    )(lens.astype(jnp.int32), hidden_states)


def _finalize_body(psc_ref, lens_ref, o_ref, *, per_seg):
    nsc = o_ref.shape[0]
    for j in range(nsc):
        s = psc_ref[pl.ds(j * per_seg, 1), :]
        for k in range(1, per_seg):
            s = s + psc_ref[pl.ds(j * per_seg + k, 1), :]
        pooled = s / lens_ref[pl.ds(j, 1), :]
        nrm = jnp.sqrt(jnp.sum(pooled * pooled))
        o_ref[pl.ds(j, 1), :] = pooled / jnp.maximum(nrm, 1e-12)


def kernel(hidden_states, extend_seq_lens):
    n = extend_seq_lens.shape[0]
    tokens, hidden = hidden_states.shape
    seg = tokens // n
    sc_segs = 4 if (n % 4 == 0 and seg % (_SUBCORES // 4) == 0) else n // 4
    per_seg = _SUBCORES // sc_segs

    partials_sc = _sc_partial_sums(hidden_states, seg, sc_segs)
    final_tc = _tc_pool(hidden_states, extend_seq_lens, seg, sc_segs)

    lens_sc = extend_seq_lens[:sc_segs].astype(jnp.float32).reshape(sc_segs, 1)
    final_sc = pl.pallas_call(
        functools.partial(_finalize_body, per_seg=per_seg),
        out_shape=jax.ShapeDtypeStruct((sc_segs, hidden), jnp.float32),
    )(partials_sc, lens_sc)
    return jnp.concatenate([final_sc, final_tc], axis=0)


# speedup vs baseline: 3.9033x; 1.0232x over previous
import functools

import jax
import jax.numpy as jnp
from jax import lax
from jax.experimental import pallas as pl
from jax.experimental.pallas import tpu as pltpu
from jax.experimental.pallas import tpu_sc as plsc

_LANES = 16
_SUBCORES = 32


def _sc_partial_sums(hidden_states, seg, sc_segs):
    hidden = hidden_states.shape[1]
    per_seg = _SUBCORES // sc_segs
    half = seg // per_seg
    chunk = 32
    nbuf = 2
    nchunks = half // chunk
    groups = hidden // _LANES
    ru = 8
    gbatch = 8

    mesh = plsc.VectorSubcoreMesh(core_axis_name="c", subcore_axis_name="s")

    @functools.partial(
        pl.kernel,
        mesh=mesh,
        out_type=jax.ShapeDtypeStruct((_SUBCORES, hidden), jnp.float32),
        scratch_types=(
            [pltpu.VMEM((chunk, hidden), jnp.float32)] * nbuf
            + [pltpu.VMEM((hidden,), jnp.float32)]
            + [pltpu.SemaphoreType.DMA] * nbuf
        ),
    )
    def sc_sums(hs_hbm, out_hbm, *refs):
        bufs = refs[:nbuf]
        acc = refs[nbuf]
        sems = refs[nbuf + 1:2 * nbuf + 1]
        c = lax.axis_index("c")
        s = lax.axis_index("s")
        g = c * 16 + s
        row0 = (g // per_seg) * seg + (g % per_seg) * half

        for j in range(groups):
            acc[pl.ds(j * _LANES, _LANES)] = jnp.zeros((_LANES,), jnp.float32)

        for b in range(nbuf - 1):
            pltpu.async_copy(
                hs_hbm.at[pl.ds(row0 + b * chunk, chunk), :], bufs[b], sems[b])

        def accumulate(buf):
            def rowstep(i, _):
                base = i * ru
                for j0 in range(0, groups, gbatch):
                    temps = []
                    for j in range(j0, j0 + gbatch):
                        sl = pl.ds(j * _LANES, _LANES)
                        vs = [buf[base + k, sl] for k in range(ru)]
                        while len(vs) > 1:
                            vs = [vs[k] + vs[k + 1]
                                  for k in range(0, len(vs), 2)]
                        temps.append(vs[0])
                    for j, v in zip(range(j0, j0 + gbatch), temps):
                        plsc.addupdate(acc.at[pl.ds(j * _LANES, _LANES)], v)
                return 0

            lax.fori_loop(0, chunk // ru, rowstep, 0)

        def outer(i, _):
            for b in range(nbuf):
                ch = i * nbuf + b
                cur, sem = bufs[b], sems[b]
                ahead = (b + nbuf - 1) % nbuf
                nxt, nsem = bufs[ahead], sems[ahead]

                @pl.when(ch + nbuf - 1 < nchunks)
                def _start_next():
                    pltpu.async_copy(
                        hs_hbm.at[pl.ds(row0 + (ch + nbuf - 1) * chunk, chunk), :],
                        nxt, nsem)

                pltpu.make_async_copy(
                    hs_hbm.at[pl.ds(row0 + ch * chunk, chunk), :], cur, sem
                ).wait()
                accumulate(cur)
            return 0

        lax.fori_loop(0, nchunks // nbuf, outer, 0)
        pltpu.sync_copy(acc, out_hbm.at[g])

    return sc_sums(hidden_states)


def _tc_pool_body(lens_ref, x_ref, o_ref, *, sc_segs):
    i = pl.program_id(0)
    s = jnp.sum(x_ref[...], axis=0, keepdims=True)
    pooled = s / lens_ref[i + sc_segs].astype(jnp.float32)
    nrm = jnp.sqrt(jnp.sum(pooled * pooled))
    o_ref[pl.ds(i + sc_segs, 1), :] = pooled / jnp.maximum(nrm, 1e-12)


def _tc_pool(hidden_states, lens, seg, sc_segs):
    n = lens.shape[0]
    hidden = hidden_states.shape[1]
    m = n - sc_segs
    grid_spec = pltpu.PrefetchScalarGridSpec(
        num_scalar_prefetch=1,
        grid=(m,),
        in_specs=[pl.BlockSpec((seg, hidden), lambda i, lens: (i + sc_segs, 0))],
        out_specs=pl.BlockSpec((n, hidden), lambda i, lens: (0, 0)),
    )
    return pl.pallas_call(
        functools.partial(_tc_pool_body, sc_segs=sc_segs),
        grid_spec=grid_spec,
        out_shape=jax.ShapeDtypeStruct((n, hidden), jnp.float32),
    )(lens.astype(jnp.int32), hidden_states)


def _finalize_body(ftc_ref, psc_ref, lens_ref, o_ref, *, per_seg, sc_segs):
    o_ref[...] = ftc_ref[...]
    for j in range(sc_segs):
        s = psc_ref[pl.ds(j * per_seg, 1), :]
        for k in range(1, per_seg):
            s = s + psc_ref[pl.ds(j * per_seg + k, 1), :]
        pooled = s / lens_ref[pl.ds(j, 1), :]
        nrm = jnp.sqrt(jnp.sum(pooled * pooled))
        o_ref[pl.ds(j, 1), :] = pooled / jnp.maximum(nrm, 1e-12)


def kernel(hidden_states, extend_seq_lens):
    n = extend_seq_lens.shape[0]
    tokens, hidden = hidden_states.shape
    seg = tokens // n
    sc_segs = 4 if (n % 4 == 0 and seg % (_SUBCORES // 4) == 0) else n // 4
    per_seg = _SUBCORES // sc_segs

    partials_sc = _sc_partial_sums(hidden_states, seg, sc_segs)
    final_tc = _tc_pool(hidden_states, extend_seq_lens, seg, sc_segs)

    lens_sc = extend_seq_lens[:sc_segs].astype(jnp.float32).reshape(sc_segs, 1)
    return pl.pallas_call(
        functools.partial(_finalize_body, per_seg=per_seg, sc_segs=sc_segs),
        out_shape=jax.ShapeDtypeStruct((n, hidden), jnp.float32),
        input_output_aliases={0: 0},
    )(final_tc, partials_sc, lens_sc)
